# Initial kernel scaffold; baseline (speedup 1.0000x reference)
import jax
import jax.numpy as jnp
from jax.experimental import pallas as pl


def kernel(x, edge_index_r0, edge_index_r1, edge_index_r2, W0, b0, W1, b1, W2, b2, Wl, bl):
    raise NotImplementedError("write your pallas kernel here")


# speedup vs baseline: 3.9709x; 3.9709x over previous
import functools

import jax
import jax.numpy as jnp
from jax import lax
from jax.experimental import pallas as pl
from jax.experimental.pallas import tpu as pltpu
from jax.experimental.pallas import tpu_sc as plsc

N = 50000
E = 200000
D = 128
H = 128
C = 16
R = 3

NW = 32
EW = E // NW
EWP = 6272
NROW = 49
EPAD = NW * EWP
NB = 6
BKT = 8448
ACC_ROWS = BKT + 128
NPAD = 50048
HHALF = 25024
CAP = EWP
HBLK = 2944
BN = 1000
GRID = N // BN

_mesh = plsc.VectorSubcoreMesh(core_axis_name="c", subcore_axis_name="s")
_params = pltpu.CompilerParams(needs_layout_passes=False)


@functools.partial(
    pl.kernel,
    mesh=_mesh,
    compiler_params=_params,
    out_type=[jax.ShapeDtypeStruct((NW, 6, 2, 1, HHALF), jnp.float32)]
    + [jax.ShapeDtypeStruct((NW, NB, NROW, 128), jnp.int32)] * 6
    + [jax.ShapeDtypeStruct((NW, 1, 128), jnp.int32)] * 3,
    scratch_types=[
        pltpu.VMEM((NROW, 128), jnp.int32),
        pltpu.VMEM((NROW, 128), jnp.int32),
        pltpu.VMEM((HHALF,), jnp.float32),
        pltpu.VMEM((NB, NROW, 128), jnp.int32),
        pltpu.VMEM((NB, NROW, 128), jnp.int32),
        pltpu.VMEM((1, 128), jnp.int32),
    ],
)
def _sc_prep(s0, d0, s1, d1, s2, d2, zeros_h, sfill_h, dfill_h,
             deg_h, bs0, bd0, bs1, bd1, bs2, bd2, c0, c1, c2,
             sv, dv, hist, sbuf, dbuf, cbuf):
    c = lax.axis_index("c")
    s = lax.axis_index("s")
    w = c * 16 + s
    srcs = [s0, s1, s2]
    dsts = [d0, d1, d2]
    bss = [bs0, bs1, bs2]
    bds = [bd0, bd1, bd2]
    cts = [c0, c1, c2]
    ones16 = jnp.ones((16,), jnp.float32)
    iota16 = lax.iota(jnp.int32, 16)

    for r in range(R):
        pltpu.sync_copy(srcs[r].at[w], sv)
        pltpu.sync_copy(dsts[r].at[w], dv)

        for a, idxv in ((0, sv), (1, dv)):
            for p in range(2):
                pltpu.sync_copy(zeros_h, hist)

                def hloop(j, carry, idxv=idxv, p=p):
                    for q in range(8):
                        vals = idxv[j, pl.ds(q * 16, 16)] - p * HHALF
                        m = (vals >= 0) & (vals < HHALF)
                        vloc = jnp.clip(vals, 0, HHALF - 1)
                        plsc.addupdate_scatter(hist, [vloc], ones16, mask=m)
                    return carry
                lax.fori_loop(0, NROW, hloop, 0)
                pltpu.sync_copy(hist, deg_h.at[w, 2 * r + a, p, 0])

        for b in range(NB):
            pltpu.sync_copy(sfill_h, sbuf.at[b])
            pltpu.sync_copy(dfill_h, dbuf.at[b])

        def bloop(j, ptrs):
            for q in range(8):
                s16 = sv[j, pl.ds(q * 16, 16)]
                d16 = dv[j, pl.ds(q * 16, 16)]
                s16 = jnp.minimum(s16, N - 1)
                b16 = sum(jnp.where(d16 >= k * BKT, 1, 0)
                          for k in range(1, NB))
                new = []
                for b in range(NB):
                    m = b16 == b
                    mi = jnp.where(m, 1, 0)
                    pos = ptrs[b] + jnp.cumsum(mi) - 1
                    row = lax.shift_right_logical(pos, 7)
                    col = lax.bitwise_and(pos, 127)
                    plsc.store_scatter(sbuf.at[b], [row, col], s16, mask=m)
                    plsc.store_scatter(dbuf.at[b], [row, col],
                                       d16 - b * BKT, mask=m)
                    new.append(ptrs[b] + jnp.sum(mi))
                ptrs = tuple(new)
            return ptrs
        z = jnp.int32(0)
        ptrs = lax.fori_loop(0, NROW, bloop, (z,) * NB)

        for b in range(NB):
            pltpu.sync_copy(sbuf.at[b], bss[r].at[w, b])
            pltpu.sync_copy(dbuf.at[b], bds[r].at[w, b])
        cvec = sum(jnp.where(iota16 == b, ptrs[b], 0) for b in range(NB))
        cbuf[0, pl.ds(0, 16)] = cvec
        pltpu.sync_copy(cbuf, cts[r].at[w])


@functools.partial(
    pl.kernel,
    mesh=_mesh,
    compiler_params=_params,
    out_type=[jax.ShapeDtypeStruct((N, 128), jnp.float32)] * R,
    scratch_types=[
        pltpu.VMEM((NROW, 128), jnp.int32),
        pltpu.VMEM((NROW, 128), jnp.int32),
        pltpu.VMEM((NW, 1, 128), jnp.int32),
        pltpu.VMEM((128, 128), jnp.float32),
        pltpu.VMEM((136, 128), jnp.float32),
        pltpu.VMEM_SHARED((ACC_ROWS, 128), jnp.float32),
        pltpu.SemaphoreType.DMA,
    ],
)
def _sc_spmm(*refs):
    t0, t1, t2 = refs[0:3]
    bs0, bd0, bs1, bd1, bs2, bd2 = refs[3:9]
    cn0, cn1, cn2 = refs[9:12]
    zeros_h = refs[12]
    outs = refs[13:16]
    srun, drun, cv, g, zbuf, acc, sem = refs[16:23]
    tables = [t0, t1, t2]
    bss = [bs0, bs1, bs2]
    bds = [bd0, bd1, bd2]
    cns = [cn0, cn1, cn2]

    c = lax.axis_index("c")
    s = lax.axis_index("s")
    pltpu.sync_copy(zeros_h, zbuf)
    for r in range(R):
        pltpu.sync_copy(cns[r], cv)
        for cc in range(3):
            for cst in range(2):
                @pl.when(c == cst)
                def _(r=r, cc=cc, cst=cst):
                    b = 3 * cst + cc
                    table = tables[r]
                    base = pl.multiple_of(s * 536, 8)

                    def zloop(z, carry):
                        off = pl.multiple_of(base + z * 136, 8)
                        pltpu.sync_copy(zbuf, acc.at[pl.ds(off, 136)])
                        return carry
                    lax.fori_loop(0, 3, zloop, 0)
                    off = pl.multiple_of(base + 408, 8)
                    pltpu.sync_copy(zbuf.at[pl.ds(0, 128)],
                                    acc.at[pl.ds(off, 128)])
                    plsc.subcore_barrier()

                    for w2 in range(2):
                        wk = s * 2 + w2
                        pltpu.sync_copy(bss[r].at[wk, b], srun)
                        pltpu.sync_copy(bds[r].at[wk, b], drun)
                        cnt = cv[wk, 0, pl.ds(0, 16)][b]
                        nblk = lax.shift_right_logical(cnt + 127, 7)

                        def sloop(j, carry, table=table):
                            pltpu.async_copy(
                                table.at[srun.at[j]], g, sem).wait()
                            pltpu.sync_copy(g, acc.at[drun.at[j]], add=True)
                            return carry
                        lax.fori_loop(0, nblk, sloop, 0)
                    plsc.subcore_barrier()

                    if b < NB - 1:
                        off = pl.multiple_of(s * 528, 8)
                        pltpu.sync_copy(
                            acc.at[pl.ds(off, 528)],
                            outs[r].at[pl.ds(b * BKT + off, 528)])
                    else:
                        @pl.when(s < 15)
                        def _():
                            off = pl.multiple_of(s * 488, 8)
                            pltpu.sync_copy(
                                acc.at[pl.ds(off, 488)],
                                outs[r].at[pl.ds(5 * BKT + off, 488)])

                        @pl.when(s == 15)
                        def _():
                            pltpu.sync_copy(
                                acc.at[pl.ds(15 * 488, 440)],
                                outs[r].at[pl.ds(5 * BKT + 15 * 488, 440)])
                    plsc.subcore_barrier()


def _norm_body(deg_ref, out_ref):
    d = jnp.sum(deg_ref[...], axis=0)
    cols = []
    for a in (0, 2, 4, 1, 3, 5):
        da = d[a][:, None]
        cols.append(jnp.where(da > 0, lax.rsqrt(jnp.maximum(da, 1.0)), 0.0))
    w = (d[0] + d[2] + d[4])[:, None]
    cols.append(w)
    cols.append(jnp.zeros((HBLK, 128 - 7), jnp.float32))
    out_ref[...] = jnp.concatenate(cols, axis=1)


def _tc_norms(deg):
    return pl.pallas_call(
        _norm_body,
        grid=(NPAD // HBLK,),
        in_specs=[pl.BlockSpec((NW, 6, HBLK), lambda i: (0, 0, i))],
        out_specs=pl.BlockSpec((HBLK, 128), lambda i: (i, 0)),
        out_shape=jax.ShapeDtypeStruct((NPAD, 128), jnp.float32),
    )(deg)


def _prep_body(x_ref, np_ref, *out_refs):
    x = x_ref[...]
    npk = np_ref[...]
    for r in range(R):
        out_refs[r][...] = x * npk[:, r:r + 1]


def _tc_prep(x, npk):
    return pl.pallas_call(
        _prep_body,
        grid=(GRID,),
        in_specs=[pl.BlockSpec((BN, 128), lambda i: (i, 0))] * 2,
        out_specs=[pl.BlockSpec((BN, 128), lambda i: (i, 0))] * R,
        out_shape=[jax.ShapeDtypeStruct((N, 128), jnp.float32)] * R,
    )(x, npk)


def _dense_body(*refs):
    y = refs[0:3]
    np_ref, w_ref, b_ref = refs[3:6]
    out_refs = refs[6:9]
    npk = np_ref[...]
    bias = b_ref[...]
    t = (bias[0] + bias[1] + bias[2])[None, :]
    for r in range(R):
        yr = y[r][...] * npk[:, 3 + r:4 + r]
        t = t + lax.dot_general(yr, w_ref[r], (((1,), (0,)), ((), ())),
                                preferred_element_type=jnp.float32)
    nrm = jnp.sqrt(jnp.sum(t * t, axis=1, keepdims=True))
    z = t / jnp.maximum(nrm, 1e-12)
    z = jnp.where(z >= 0, z, 0.01 * z)
    for r in range(R):
        out_refs[r][...] = z * npk[:, r:r + 1]


def _tc_dense(ys, npk, W, b):
    return pl.pallas_call(
        _dense_body,
        grid=(GRID,),
        in_specs=(
            [pl.BlockSpec((BN, 128), lambda i: (i, 0))] * (R + 1)
            + [pl.BlockSpec((R, D, H), lambda i: (0, 0, 0)),
               pl.BlockSpec((R, H), lambda i: (0, 0))]
        ),
        out_specs=[pl.BlockSpec((BN, 128), lambda i: (i, 0))] * R,
        out_shape=[jax.ShapeDtypeStruct((N, 128), jnp.float32)] * R,
    )(*ys, npk, W, b)


def _final_body(*refs):
    y = refs[0:3]
    np_ref, w_ref, b_ref, wl_ref, bl_ref = refs[3:8]
    out_ref = refs[8]
    acc_ref = refs[9]
    i = pl.program_id(0)
    npk = np_ref[...]
    bias = b_ref[...]
    t = (bias[0] + bias[1] + bias[2])[None, :]
    for r in range(R):
        yr = y[r][...] * npk[:, 3 + r:4 + r]
        t = t + lax.dot_general(yr, w_ref[r], (((1,), (0,)), ((), ())),
                                preferred_element_type=jnp.float32)
    part = jnp.sum(t * npk[:, 6:7], axis=0, keepdims=True)

    @pl.when(i == 0)
    def _():
        acc_ref[...] = jnp.zeros((8, 128), jnp.float32)

    acc_ref[0:1, :] = acc_ref[0:1, :] + part

    @pl.when(i == GRID - 1)
    def _():
        hg = acc_ref[0:1, :] * (1.0 / N)
        logits = lax.dot_general(hg, wl_ref[...], (((1,), (1,)), ((), ())),
                                 preferred_element_type=jnp.float32)
        out_ref[...] = jax.nn.sigmoid(logits + bl_ref[...])


def _tc_final(ys, npk, W, b, Wl, bl):
    return pl.pallas_call(
        _final_body,
        grid=(GRID,),
        in_specs=(
            [pl.BlockSpec((BN, 128), lambda i: (i, 0))] * (R + 1)
            + [pl.BlockSpec((R, D, H), lambda i: (0, 0, 0)),
               pl.BlockSpec((R, H), lambda i: (0, 0)),
               pl.BlockSpec((C, H), lambda i: (0, 0)),
               pl.BlockSpec((1, C), lambda i: (0, 0))]
        ),
        out_specs=pl.BlockSpec((1, C), lambda i: (0, 0)),
        out_shape=jax.ShapeDtypeStruct((1, C), jnp.float32),
        scratch_shapes=[pltpu.VMEM((8, 128), jnp.float32)],
    )(*ys, npk, W, b, Wl, bl)


def _prep_edges(e):
    pad = EPAD - E
    src = jnp.concatenate([e[0], jnp.full((pad,), N, jnp.int32)])
    dst = jnp.concatenate([e[1], jnp.full((pad,), N, jnp.int32)])
    return (src.reshape(NW, NROW, 128), dst.reshape(NW, NROW, 128))


def kernel(x, edge_index_r0, edge_index_r1, edge_index_r2,
           W0, b0, W1, b1, W2, b2, Wl, bl):
    s0, d0 = _prep_edges(edge_index_r0)
    s1, d1 = _prep_edges(edge_index_r1)
    s2, d2 = _prep_edges(edge_index_r2)

    zeros_hist = jnp.zeros((HHALF,), jnp.float32)
    sfill = ((jnp.arange(EWP, dtype=jnp.int32) * 131) % N).reshape(NROW, 128)
    dfill = jnp.full((NROW, 128), BKT, jnp.int32)
    zeros136 = jnp.zeros((136, 128), jnp.float32)

    prep = _sc_prep(s0, d0, s1, d1, s2, d2, zeros_hist, sfill, dfill)
    deg = prep[0].reshape(NW, 6, NPAD)
    buckets = prep[1:7]
    counts = prep[7:10]

    npk = _tc_norms(deg)

    spmm = lambda tabs: _sc_spmm(*tabs, *buckets, *counts, zeros136)
    ch = _tc_prep(x, npk)
    y = spmm(ch)
    ch = _tc_dense(y, npk, W0, b0)
    y = spmm(ch)
    ch = _tc_dense(y, npk, W1, b1)
    y = spmm(ch)
    return _tc_final(y, npk, W2, b2, Wl, bl.reshape(1, C))


# speedup vs baseline: 4.3010x; 1.0831x over previous
import functools

import jax
import jax.numpy as jnp
from jax import lax
from jax.experimental import pallas as pl
from jax.experimental.pallas import tpu as pltpu
from jax.experimental.pallas import tpu_sc as plsc

N = 50000
E = 200000
D = 128
H = 128
C = 16
R = 3

NW = 32
EW = E // NW
EWP = 6272
NROW = 49
EPAD = NW * EWP
NB = 6
BKT = 8448
ACC_ROWS = BKT + 128
NPAD = 50048
HHALF = 25024
CAP = EWP
HBLK = 2944
BN = 1000
GRID = N // BN

_mesh = plsc.VectorSubcoreMesh(core_axis_name="c", subcore_axis_name="s")
_params = pltpu.CompilerParams(needs_layout_passes=False)


@functools.partial(
    pl.kernel,
    mesh=_mesh,
    compiler_params=_params,
    out_type=[jax.ShapeDtypeStruct((NW, 6, 2, 1, HHALF), jnp.float32)]
    + [jax.ShapeDtypeStruct((NW, NB, NROW, 128), jnp.int32)] * 6
    + [jax.ShapeDtypeStruct((NW, 1, 128), jnp.int32)] * 3,
    scratch_types=[
        pltpu.VMEM((NROW, 128), jnp.int32),
        pltpu.VMEM((NROW, 128), jnp.int32),
        pltpu.VMEM((HHALF,), jnp.float32),
        pltpu.VMEM((NB, NROW, 128), jnp.int32),
        pltpu.VMEM((NB, NROW, 128), jnp.int32),
        pltpu.VMEM((1, 128), jnp.int32),
    ],
)
def _sc_prep(s0, d0, s1, d1, s2, d2, zeros_h, sfill_h, dfill_h,
             deg_h, bs0, bd0, bs1, bd1, bs2, bd2, c0, c1, c2,
             sv, dv, hist, sbuf, dbuf, cbuf):
    c = lax.axis_index("c")
    s = lax.axis_index("s")
    w = c * 16 + s
    srcs = [s0, s1, s2]
    dsts = [d0, d1, d2]
    bss = [bs0, bs1, bs2]
    bds = [bd0, bd1, bd2]
    cts = [c0, c1, c2]
    ones16 = jnp.ones((16,), jnp.float32)
    iota16 = lax.iota(jnp.int32, 16)

    for r in range(R):
        pltpu.sync_copy(srcs[r].at[w], sv)
        pltpu.sync_copy(dsts[r].at[w], dv)

        for a, idxv in ((0, sv), (1, dv)):
            for p in range(2):
                pltpu.sync_copy(zeros_h, hist)

                def hloop(j, carry, idxv=idxv, p=p):
                    for q in range(8):
                        vals = idxv[j, pl.ds(q * 16, 16)] - p * HHALF
                        m = (vals >= 0) & (vals < HHALF)
                        vloc = jnp.clip(vals, 0, HHALF - 1)
                        plsc.addupdate_scatter(hist, [vloc], ones16, mask=m)
                    return carry
                lax.fori_loop(0, NROW, hloop, 0)
                pltpu.sync_copy(hist, deg_h.at[w, 2 * r + a, p, 0])

        for b in range(NB):
            pltpu.sync_copy(sfill_h, sbuf.at[b])
            pltpu.sync_copy(dfill_h, dbuf.at[b])

        def bloop(j, ptrs):
            for q in range(8):
                s16 = sv[j, pl.ds(q * 16, 16)]
                d16 = dv[j, pl.ds(q * 16, 16)]
                s16 = jnp.minimum(s16, N - 1)
                b16 = sum(jnp.where(d16 >= k * BKT, 1, 0)
                          for k in range(1, NB))
                new = []
                for b in range(NB):
                    m = b16 == b
                    mi = jnp.where(m, 1, 0)
                    pos = ptrs[b] + jnp.cumsum(mi) - 1
                    row = lax.shift_right_logical(pos, 7)
                    col = lax.bitwise_and(pos, 127)
                    plsc.store_scatter(sbuf.at[b], [row, col], s16, mask=m)
                    plsc.store_scatter(dbuf.at[b], [row, col],
                                       d16 - b * BKT, mask=m)
                    new.append(ptrs[b] + jnp.sum(mi))
                ptrs = tuple(new)
            return ptrs
        z = jnp.int32(0)
        ptrs = lax.fori_loop(0, NROW, bloop, (z,) * NB)

        for b in range(NB):
            pltpu.sync_copy(sbuf.at[b], bss[r].at[w, b])
            pltpu.sync_copy(dbuf.at[b], bds[r].at[w, b])
        cvec = sum(jnp.where(iota16 == b, ptrs[b], 0) for b in range(NB))
        cbuf[0, pl.ds(0, 16)] = cvec
        pltpu.sync_copy(cbuf, cts[r].at[w])


@functools.partial(
    pl.kernel,
    mesh=_mesh,
    compiler_params=_params,
    out_type=[jax.ShapeDtypeStruct((N, 128), jnp.float32)] * R,
    scratch_types=[
        pltpu.VMEM((NROW, 128), jnp.int32),
        pltpu.VMEM((NROW, 128), jnp.int32),
        pltpu.VMEM((2, 1, 128), jnp.int32),
        pltpu.VMEM((128, 128), jnp.float32),
        pltpu.VMEM((128, 128), jnp.float32),
        pltpu.VMEM_SHARED((ACC_ROWS, 128), jnp.float32),
        pltpu.SemaphoreType.DMA,
        pltpu.SemaphoreType.DMA,
    ],
)
def _sc_spmm(*refs):
    t0, t1, t2 = refs[0:3]
    bs0, bd0, bs1, bd1, bs2, bd2 = refs[3:9]
    cn0, cn1, cn2 = refs[9:12]
    zeros_h = refs[12]
    outs = refs[13:16]
    srun, drun, cv, g0, g1, acc, sem0, sem1 = refs[16:24]
    tables = [t0, t1, t2]
    bss = [bs0, bs1, bs2]
    bds = [bd0, bd1, bd2]
    cns = [cn0, cn1, cn2]

    c = lax.axis_index("c")
    s = lax.axis_index("s")
    for r in range(R):
        pltpu.sync_copy(cns[r].at[pl.ds(s * 2, 2)], cv)
        for cc in range(3):
            for cst in range(2):
                @pl.when(c == cst)
                def _(r=r, cc=cc, cst=cst):
                    b = 3 * cst + cc
                    table = tables[r]
                    base = pl.multiple_of(s * 536, 8)

                    pltpu.sync_copy(zeros_h, g0)

                    def zloop(z, carry):
                        off = pl.multiple_of(base + z * 128, 8)
                        pltpu.sync_copy(g0, acc.at[pl.ds(off, 128)])
                        return carry
                    lax.fori_loop(0, 4, zloop, 0)
                    off = pl.multiple_of(base + 512, 8)
                    pltpu.sync_copy(g0.at[pl.ds(0, 24)],
                                    acc.at[pl.ds(off, 24)])
                    plsc.subcore_barrier()

                    for w2 in range(2):
                        pltpu.sync_copy(bss[r].at[s * 2 + w2, b], srun)
                        pltpu.sync_copy(bds[r].at[s * 2 + w2, b], drun)
                        cnt = cv[w2, 0, pl.ds(0, 16)][b]
                        nblk = lax.shift_right_logical(cnt + 127, 7)

                        def start0(j, table=table):
                            pltpu.make_async_copy(
                                table.at[srun.at[j]], g0, sem0).start()

                        def start1(j, table=table):
                            pltpu.make_async_copy(
                                table.at[srun.at[j]], g1, sem1).start()

                        @pl.when(nblk > 0)
                        def _():
                            start0(0)

                        def ploop(i, carry, table=table, nblk=nblk):
                            b0 = 2 * i
                            b1 = b0 + 1
                            pltpu.make_async_copy(
                                table.at[srun.at[b0]], g0, sem0).wait()

                            @pl.when(b1 < nblk)
                            def _():
                                start1(b1)
                            pltpu.sync_copy(g0, acc.at[drun.at[b0]],
                                            add=True)

                            @pl.when(b1 < nblk)
                            def _():
                                pltpu.make_async_copy(
                                    table.at[srun.at[b1]], g1, sem1).wait()

                                @pl.when(b1 + 1 < nblk)
                                def _():
                                    start0(b1 + 1)
                                pltpu.sync_copy(g1, acc.at[drun.at[b1]],
                                                add=True)
                            return carry
                        lax.fori_loop(
                            0, lax.shift_right_logical(nblk + 1, 1),
                            ploop, 0)
                    plsc.subcore_barrier()

                    if b < NB - 1:
                        off = pl.multiple_of(s * 528, 8)
                        pltpu.sync_copy(
                            acc.at[pl.ds(off, 528)],
                            outs[r].at[pl.ds(b * BKT + off, 528)])
                    else:
                        @pl.when(s < 15)
                        def _():
                            off = pl.multiple_of(s * 488, 8)
                            pltpu.sync_copy(
                                acc.at[pl.ds(off, 488)],
                                outs[r].at[pl.ds(5 * BKT + off, 488)])

                        @pl.when(s == 15)
                        def _():
                            pltpu.sync_copy(
                                acc.at[pl.ds(15 * 488, 440)],
                                outs[r].at[pl.ds(5 * BKT + 15 * 488, 440)])
                    plsc.subcore_barrier()


def _norm_body(deg_ref, out_ref):
    d = jnp.sum(deg_ref[...], axis=0)
    cols = []
    for a in (0, 2, 4, 1, 3, 5):
        da = d[a][:, None]
        cols.append(jnp.where(da > 0, lax.rsqrt(jnp.maximum(da, 1.0)), 0.0))
    w = (d[0] + d[2] + d[4])[:, None]
    cols.append(w)
    cols.append(jnp.zeros((HBLK, 128 - 7), jnp.float32))
    out_ref[...] = jnp.concatenate(cols, axis=1)


def _tc_norms(deg):
    return pl.pallas_call(
        _norm_body,
        grid=(NPAD // HBLK,),
        in_specs=[pl.BlockSpec((NW, 6, HBLK), lambda i: (0, 0, i))],
        out_specs=pl.BlockSpec((HBLK, 128), lambda i: (i, 0)),
        out_shape=jax.ShapeDtypeStruct((NPAD, 128), jnp.float32),
    )(deg)


def _prep_body(x_ref, np_ref, *out_refs):
    x = x_ref[...]
    npk = np_ref[...]
    for r in range(R):
        out_refs[r][...] = x * npk[:, r:r + 1]


def _tc_prep(x, npk):
    return pl.pallas_call(
        _prep_body,
        grid=(GRID,),
        in_specs=[pl.BlockSpec((BN, 128), lambda i: (i, 0))] * 2,
        out_specs=[pl.BlockSpec((BN, 128), lambda i: (i, 0))] * R,
        out_shape=[jax.ShapeDtypeStruct((N, 128), jnp.float32)] * R,
    )(x, npk)


def _dense_body(*refs):
    y = refs[0:3]
    np_ref, w_ref, b_ref = refs[3:6]
    out_refs = refs[6:9]
    npk = np_ref[...]
    bias = b_ref[...]
    t = (bias[0] + bias[1] + bias[2])[None, :]
    for r in range(R):
        yr = y[r][...] * npk[:, 3 + r:4 + r]
        t = t + lax.dot_general(yr, w_ref[r], (((1,), (0,)), ((), ())),
                                preferred_element_type=jnp.float32)
    nrm = jnp.sqrt(jnp.sum(t * t, axis=1, keepdims=True))
    z = t / jnp.maximum(nrm, 1e-12)
    z = jnp.where(z >= 0, z, 0.01 * z)
    for r in range(R):
        out_refs[r][...] = z * npk[:, r:r + 1]


def _tc_dense(ys, npk, W, b):
    return pl.pallas_call(
        _dense_body,
        grid=(GRID,),
        in_specs=(
            [pl.BlockSpec((BN, 128), lambda i: (i, 0))] * (R + 1)
            + [pl.BlockSpec((R, D, H), lambda i: (0, 0, 0)),
               pl.BlockSpec((R, H), lambda i: (0, 0))]
        ),
        out_specs=[pl.BlockSpec((BN, 128), lambda i: (i, 0))] * R,
        out_shape=[jax.ShapeDtypeStruct((N, 128), jnp.float32)] * R,
    )(*ys, npk, W, b)


def _final_body(*refs):
    y = refs[0:3]
    np_ref, w_ref, b_ref, wl_ref, bl_ref = refs[3:8]
    out_ref = refs[8]
    acc_ref = refs[9]
    i = pl.program_id(0)
    npk = np_ref[...]
    bias = b_ref[...]
    t = (bias[0] + bias[1] + bias[2])[None, :]
    for r in range(R):
        yr = y[r][...] * npk[:, 3 + r:4 + r]
        t = t + lax.dot_general(yr, w_ref[r], (((1,), (0,)), ((), ())),
                                preferred_element_type=jnp.float32)
    part = jnp.sum(t * npk[:, 6:7], axis=0, keepdims=True)

    @pl.when(i == 0)
    def _():
        acc_ref[...] = jnp.zeros((8, 128), jnp.float32)

    acc_ref[0:1, :] = acc_ref[0:1, :] + part

    @pl.when(i == GRID - 1)
    def _():
        hg = acc_ref[0:1, :] * (1.0 / N)
        logits = lax.dot_general(hg, wl_ref[...], (((1,), (1,)), ((), ())),
                                 preferred_element_type=jnp.float32)
        out_ref[...] = jax.nn.sigmoid(logits + bl_ref[...])


def _tc_final(ys, npk, W, b, Wl, bl):
    return pl.pallas_call(
        _final_body,
        grid=(GRID,),
        in_specs=(
            [pl.BlockSpec((BN, 128), lambda i: (i, 0))] * (R + 1)
            + [pl.BlockSpec((R, D, H), lambda i: (0, 0, 0)),
               pl.BlockSpec((R, H), lambda i: (0, 0)),
               pl.BlockSpec((C, H), lambda i: (0, 0)),
               pl.BlockSpec((1, C), lambda i: (0, 0))]
        ),
        out_specs=pl.BlockSpec((1, C), lambda i: (0, 0)),
        out_shape=jax.ShapeDtypeStruct((1, C), jnp.float32),
        scratch_shapes=[pltpu.VMEM((8, 128), jnp.float32)],
    )(*ys, npk, W, b, Wl, bl)


def _prep_edges(e):
    pad = EPAD - E
    src = jnp.concatenate([e[0], jnp.full((pad,), N, jnp.int32)])
    dst = jnp.concatenate([e[1], jnp.full((pad,), N, jnp.int32)])
    return (src.reshape(NW, NROW, 128), dst.reshape(NW, NROW, 128))


def kernel(x, edge_index_r0, edge_index_r1, edge_index_r2,
           W0, b0, W1, b1, W2, b2, Wl, bl):
    s0, d0 = _prep_edges(edge_index_r0)
    s1, d1 = _prep_edges(edge_index_r1)
    s2, d2 = _prep_edges(edge_index_r2)

    zeros_hist = jnp.zeros((HHALF,), jnp.float32)
    sfill = ((jnp.arange(EWP, dtype=jnp.int32) * 131) % N).reshape(NROW, 128)
    dfill = jnp.full((NROW, 128), BKT, jnp.int32)
    zeros128 = jnp.zeros((128, 128), jnp.float32)

    prep = _sc_prep(s0, d0, s1, d1, s2, d2, zeros_hist, sfill, dfill)
    deg = prep[0].reshape(NW, 6, NPAD)
    buckets = prep[1:7]
    counts = prep[7:10]

    npk = _tc_norms(deg)

    spmm = lambda tabs: _sc_spmm(*tabs, *buckets, *counts, zeros128)
    ch = _tc_prep(x, npk)
    y = spmm(ch)
    ch = _tc_dense(y, npk, W0, b0)
    y = spmm(ch)
    ch = _tc_dense(y, npk, W1, b1)
    y = spmm(ch)
    return _tc_final(y, npk, W2, b2, Wl, bl.reshape(1, C))
